# baseline (speedup 1.0000x reference)
import functools

import jax
import jax.numpy as jnp
from jax import lax
from jax.experimental import pallas as pl
from jax.experimental.pallas import tpu as pltpu
from jax.experimental.pallas import tpu_sc as plsc

NC = 2
NS = 16
NW = NC * NS
C = 128
B = 4
SB = 8
DC = 8
DG = 32


@functools.lru_cache(maxsize=None)
def _make_scatter(n_pad: int, nch0: int, nch1: int, d: int):
    dg = min(d, DG)
    ng = d // dg
    assert dg * ng == d
    assert nch0 % (2 * SB) == 0 and nch1 % (2 * SB) == 0
    nchmax = max(nch0, nch1)
    rows_pt = n_pad // NS
    assert rows_pt == B * C + C
    mesh = plsc.VectorSubcoreMesh(
        core_axis_name="c", subcore_axis_name="s",
        num_cores=NC, num_subcores=NS)

    @functools.partial(
        pl.kernel,
        out_type=jax.ShapeDtypeStruct((NC, ng, n_pad, dg), jnp.float32),
        mesh=mesh,
        scratch_types=[
            pltpu.VMEM((nchmax + DC, C), jnp.int32),
            pltpu.VMEM((nchmax + DC, C), jnp.int32),
            pltpu.VMEM((SB * C, dg), jnp.float32),
            pltpu.VMEM((SB * C, dg), jnp.float32),
            pltpu.VMEM_SHARED((n_pad, dg), jnp.float32),
            pltpu.VMEM_SHARED((n_pad, dg), jnp.float32),
            pltpu.SemaphoreType.DMA,
        ],
        compiler_params=pltpu.CompilerParams(use_tc_tiling_on_sc=False),
    )
    def scat(g_hbm, srci_hbm, dsti_hbm, zrow_hbm, out_hbm,
             srci, dsti, mA, mB, acc, g_sp, s):
        cid = lax.axis_index("c")
        sid = lax.axis_index("s")
        wid = cid * NS + sid
        nbh = jnp.where(cid == 0, nch0 // (2 * SB), nch1 // (2 * SB))
        base = sid * rows_pt

        def gath(bi, buf):
            return [pltpu.make_async_copy(
                        g_sp.at[srci.at[SB * bi + k]],
                        buf.at[pl.ds(k * C, C)], s)
                    for k in range(SB)]

        def scab(bi, buf):
            return [pltpu.make_async_copy(
                        buf.at[pl.ds(k * C, C)],
                        acc.at[dsti.at[SB * bi + k]], s)
                    for k in range(SB)]

        pltpu.sync_copy(srci_hbm.at[wid], srci)
        pltpu.sync_copy(dsti_hbm.at[wid], dsti)

        for h in range(ng):
            mAs = mA.at[pl.ds(0, B * C)]
            pltpu.sync_copy(g_hbm.at[h, pl.ds(base, B * C)], mAs)
            pltpu.sync_copy(mAs, g_sp.at[pl.ds(base, B * C)])
            pltpu.sync_copy(g_hbm.at[h, pl.ds(base + B * C, C)],
                            mB.at[pl.ds(0, C)])
            pltpu.sync_copy(mB.at[pl.ds(0, C)],
                            g_sp.at[pl.ds(base + B * C, C)])
            pltpu.sync_copy(zrow_hbm, mAs)
            pltpu.sync_copy(mAs, acc.at[pl.ds(base, B * C)])
            pltpu.sync_copy(mA.at[pl.ds(0, C)], acc.at[pl.ds(base + B * C, C)])
            plsc.subcore_barrier()

            for c in gath(0, mA):
                c.start()
            for c in gath(0, mA):
                c.wait()

            def body(i, carry):
                for (bi, cur, oth) in ((2 * i, mA, mB), (2 * i + 1, mB, mA)):
                    for c in gath(bi + 1, oth):
                        c.start()
                    for c in scab(bi, cur):
                        c.start(add=True)
                    for c in gath(bi + 1, oth):
                        c.wait()
                    for c in scab(bi, cur):
                        c.wait()
                return carry

            lax.fori_loop(0, nbh, body, 0)
            plsc.subcore_barrier()
            pltpu.sync_copy(acc.at[pl.ds(base, B * C)], mA.at[pl.ds(0, B * C)])
            pltpu.sync_copy(mA.at[pl.ds(0, B * C)],
                            out_hbm.at[cid, h, pl.ds(base, B * C)])
            pltpu.sync_copy(acc.at[pl.ds(base + B * C, C)], mB.at[pl.ds(0, C)])
            pltpu.sync_copy(mB.at[pl.ds(0, C)],
                            out_hbm.at[cid, h, pl.ds(base + B * C, C)])

    return scat


@functools.lru_cache(maxsize=None)
def _make_deg(n_pad: int, nch0: int, nch1: int):
    d = 16
    rows_pt = n_pad // NS
    assert rows_pt == B * C + C
    assert nch0 % 8 == 0 and nch1 % 8 == 0 and min(nch0, nch1) >= 24
    nchmax = max(nch0, nch1)
    mesh = plsc.VectorSubcoreMesh(
        core_axis_name="c", subcore_axis_name="s",
        num_cores=NC, num_subcores=NS)

    @functools.partial(
        pl.kernel,
        out_type=jax.ShapeDtypeStruct((NC, n_pad, d), jnp.float32),
        mesh=mesh,
        scratch_types=[
            pltpu.VMEM((nchmax + DC, C), jnp.int32),
            pltpu.VMEM((C, d), jnp.float32),
            pltpu.VMEM((B * C, d), jnp.float32),
            pltpu.VMEM_SHARED((n_pad, d), jnp.float32),
            pltpu.SemaphoreType.DMA,
        ],
        compiler_params=pltpu.CompilerParams(use_tc_tiling_on_sc=False),
    )
    def deg(dsti_hbm, ones_hbm, zrow_hbm, out_hbm, dsti, ones, stage, acc, ss):
        cid = lax.axis_index("c")
        sid = lax.axis_index("s")
        wid = cid * NS + sid
        nchc = jnp.where(cid == 0, nch0, nch1)
        base = sid * rows_pt

        def sca(j):
            return pltpu.make_async_copy(ones, acc.at[dsti.at[j]], ss)

        pltpu.sync_copy(dsti_hbm.at[wid], dsti)
        pltpu.sync_copy(ones_hbm, ones)
        pltpu.sync_copy(zrow_hbm, stage)
        pltpu.sync_copy(stage, acc.at[pl.ds(base, B * C)])
        pltpu.sync_copy(stage.at[pl.ds(0, C)], acc.at[pl.ds(base + B * C, C)])
        plsc.subcore_barrier()

        for j in range(16):
            sca(j).start(add=True)

        def body(i, carry):
            j8 = 8 * i
            for k in range(8):
                sca(j8 + k).wait()
            for k in range(8):
                sca(j8 + 16 + k).start(add=True)
            return carry

        lax.fori_loop(0, (nchc - 16) // 8, body, 0)

        def drain(i, carry):
            for k in range(8):
                sca(nchc - 16 + 8 * i + k).wait()
            return carry

        lax.fori_loop(0, 2, drain, 0)
        plsc.subcore_barrier()
        pltpu.sync_copy(acc.at[pl.ds(base, B * C)], stage)
        pltpu.sync_copy(stage, out_hbm.at[cid, pl.ds(base, B * C)])
        pltpu.sync_copy(acc.at[pl.ds(base + B * C, C)], stage.at[pl.ds(0, C)])
        pltpu.sync_copy(stage.at[pl.ds(0, C)],
                        out_hbm.at[cid, pl.ds(base + B * C, C)])

    return deg


def _split(gfull, n_pad, d):
    dg = min(d, DG)
    ng = d // dg
    if ng == 1:
        return gfull.reshape(1, n_pad, dg)
    return jnp.stack([gfull[:, h * dg:(h + 1) * dg] for h in range(ng)])


def _joined(acc_ref, g_ref, d):
    dg = min(d, DG)
    ng = d // dg
    parts = [acc_ref[0, h] + acc_ref[1, h] + g_ref[h] for h in range(ng)]
    if ng == 1:
        return parts[0]
    return jnp.concatenate(parts, axis=1)


@functools.lru_cache(maxsize=None)
def _make_tc_mm(n_pad: int, in_dim: int, hid: int):
    def body(x_ref, w_ref, m_ref):
        m_ref[...] = jnp.dot(x_ref[...], w_ref[...],
                             preferred_element_type=jnp.float32)

    return pl.pallas_call(
        body,
        out_shape=jax.ShapeDtypeStruct((n_pad, hid), jnp.float32),
    )


_TC_GRID = 8


@functools.lru_cache(maxsize=None)
def _make_tc_first(n_pad: int, hid: int):
    dg = min(hid, DG)
    ng = hid // dg
    r = n_pad // _TC_GRID

    def body(degp_ref, m_ref, g_ref, dinv_ref):
        deg = degp_ref[0, :, 0:1] + degp_ref[1, :, 0:1] + 1.0
        dinv = lax.rsqrt(deg)
        dinv_ref[...] = jnp.broadcast_to(dinv, (r, 8))
        g_ref[...] = _split(m_ref[...] * dinv, r, hid)

    return pl.pallas_call(
        body,
        grid=(_TC_GRID,),
        in_specs=[
            pl.BlockSpec((NC, r, 16), lambda i: (0, i, 0)),
            pl.BlockSpec((r, hid), lambda i: (i, 0)),
        ],
        out_specs=(
            pl.BlockSpec((ng, r, dg), lambda i: (0, i, 0)),
            pl.BlockSpec((r, 8), lambda i: (i, 0)),
        ),
        out_shape=(
            jax.ShapeDtypeStruct((ng, n_pad, dg), jnp.float32),
            jax.ShapeDtypeStruct((n_pad, 8), jnp.float32),
        ),
    )


@functools.lru_cache(maxsize=None)
def _make_tc_next(n_pad: int, d_in: int, d_out: int):
    dgi = min(d_in, DG)
    ngi = d_in // dgi
    dgo = min(d_out, DG)
    ngo = d_out // dgo
    r = n_pad // _TC_GRID

    def body(acc_ref, g_ref, dinv_ref, b_ref, w_ref, o_ref):
        dv = dinv_ref[:, 0:1]
        conv = dv * _joined(acc_ref, g_ref, d_in) + b_ref[...]
        h = jnp.maximum(conv, 0.0)
        m = jnp.dot(h, w_ref[...], preferred_element_type=jnp.float32)
        o_ref[...] = _split(dv * m, r, d_out)

    return pl.pallas_call(
        body,
        grid=(_TC_GRID,),
        in_specs=[
            pl.BlockSpec((NC, ngi, r, dgi), lambda i: (0, 0, i, 0)),
            pl.BlockSpec((ngi, r, dgi), lambda i: (0, i, 0)),
            pl.BlockSpec((r, 8), lambda i: (i, 0)),
            pl.BlockSpec((1, d_in), lambda i: (0, 0)),
            pl.BlockSpec((d_in, d_out), lambda i: (0, 0)),
        ],
        out_specs=pl.BlockSpec((ngo, r, dgo), lambda i: (0, i, 0)),
        out_shape=jax.ShapeDtypeStruct((ngo, n_pad, dgo), jnp.float32),
    )


@functools.lru_cache(maxsize=None)
def _make_tc_final(n_pad: int, d: int, n_out: int, emb: int):
    def body(acc_ref, g_ref, dinv_ref, b_ref, o_ref):
        dv = dinv_ref[:, 0:1]
        full = dv * _joined(acc_ref, g_ref, d) + b_ref[...]
        o_ref[...] = full[:n_out, :emb]

    return pl.pallas_call(
        body,
        out_shape=jax.ShapeDtypeStruct((n_out, emb), jnp.float32),
    )


def kernel(x, edge_index, W1, b1, W2, b2, W3, b3):
    n, in_dim = x.shape
    e = edge_index.shape[1]
    hid = W1.shape[1]
    emb = W3.shape[1]

    n_pad = -(-n // (NS * (B + 1) * C)) * (NS * (B + 1) * C)
    ct = -(-e // C)
    nch0 = max(2 * SB, -(-ct // (NW * 2 * SB)) * (2 * SB))
    nch1 = nch0
    nchmax = max(nch0, nch1)
    e_pad = NS * (nch0 + nch1) * C
    trash = jnp.int32(n)

    src = edge_index[0]
    dst = edge_index[1]

    def _layout(arr):
        a = jnp.concatenate(
            [arr, jnp.full((e_pad - e,), trash, dtype=jnp.int32)])
        c0 = a[:NS * nch0 * C].reshape(NS, nch0, C)
        c1 = a[NS * nch0 * C:].reshape(NS, nch1, C)
        c0 = jnp.pad(c0, ((0, 0), (0, nchmax + DC - nch0), (0, 0)),
                     constant_values=n)
        c1 = jnp.pad(c1, ((0, 0), (0, nchmax + DC - nch1), (0, 0)),
                     constant_values=n)
        return jnp.concatenate([c0, c1])

    srcp = _layout(src)
    dstp = _layout(dst)

    x_pad = jnp.pad(x, ((0, n_pad - n), (0, 0)))
    ones16 = jnp.ones((C, 16), dtype=jnp.float32)
    z16 = jnp.zeros((B * C, 16), dtype=jnp.float32)
    z32 = jnp.zeros((B * C, DG), dtype=jnp.float32)
    emb_p = 16
    W3p = jnp.pad(W3, ((0, 0), (0, emb_p - emb)))
    b3p = jnp.pad(b3, (0, emb_p - emb)).reshape(1, emb_p)
    b1r = b1.reshape(1, hid)
    b2r = b2.reshape(1, hid)

    scat16 = _make_scatter(n_pad, nch0, nch1, emb_p)
    scath = _make_scatter(n_pad, nch0, nch1, hid)

    degp = _make_deg(n_pad, nch0, nch1)(dstp, ones16, z16)
    m1 = _make_tc_mm(n_pad, in_dim, hid)(x_pad, W1)
    g1, dinv = _make_tc_first(n_pad, hid)(degp, m1)
    acc1 = scath(g1, srcp, dstp, z32)
    g2 = _make_tc_next(n_pad, hid, hid)(acc1, g1, dinv, b1r, W2)
    acc2 = scath(g2, srcp, dstp, z32)
    g3 = _make_tc_next(n_pad, hid, emb_p)(acc2, g2, dinv, b2r, W3p)
    acc3 = scat16(g3, srcp, dstp, z16)
    return _make_tc_final(n_pad, emb_p, n, emb)(acc3, g3, dinv, b3p)


# speedup vs baseline: 1.0432x; 1.0432x over previous
import functools

import jax
import jax.numpy as jnp
from jax import lax
from jax.experimental import pallas as pl
from jax.experimental.pallas import tpu as pltpu
from jax.experimental.pallas import tpu_sc as plsc

NC = 2
NS = 16
NW = NC * NS
C = 128
B = 4
DC = 4
DG = 32


@functools.lru_cache(maxsize=None)
def _make_scatter(n_pad: int, nchunk: int, d: int):
    dg = min(d, DG)
    ng = d // dg
    assert dg * ng == d
    assert nchunk % (2 * B) == 0
    nb = nchunk // B
    rows_pt = n_pad // NS
    assert rows_pt == B * C + C
    mesh = plsc.VectorSubcoreMesh(
        core_axis_name="c", subcore_axis_name="s",
        num_cores=NC, num_subcores=NS)

    @functools.partial(
        pl.kernel,
        out_type=jax.ShapeDtypeStruct((NC, ng, n_pad, dg), jnp.float32),
        mesh=mesh,
        scratch_types=[
            pltpu.VMEM((nchunk + DC, C), jnp.int32),
            pltpu.VMEM((nchunk + DC, C), jnp.int32),
            pltpu.VMEM((B * C, dg), jnp.float32),
            pltpu.VMEM((B * C, dg), jnp.float32),
            pltpu.VMEM_SHARED((n_pad, dg), jnp.float32),
            pltpu.VMEM_SHARED((n_pad, dg), jnp.float32),
            pltpu.SemaphoreType.DMA,
        ],
        compiler_params=pltpu.CompilerParams(use_tc_tiling_on_sc=False),
    )
    def scat(g_hbm, srci_hbm, dsti_hbm, zrow_hbm, out_hbm,
             srci, dsti, mA, mB, acc, g_sp, s):
        cid = lax.axis_index("c")
        sid = lax.axis_index("s")
        wid = sid * NC + cid
        base = sid * rows_pt

        def gath(bi, buf):
            return [pltpu.make_async_copy(
                        g_sp.at[srci.at[B * bi + k]],
                        buf.at[pl.ds(k * C, C)], s)
                    for k in range(B)]

        def scab(bi, buf):
            return [pltpu.make_async_copy(
                        buf.at[pl.ds(k * C, C)],
                        acc.at[dsti.at[B * bi + k]], s)
                    for k in range(B)]

        pltpu.sync_copy(srci_hbm.at[wid], srci)
        pltpu.sync_copy(dsti_hbm.at[wid], dsti)

        for h in range(ng):
            pltpu.sync_copy(g_hbm.at[h, pl.ds(base, B * C)], mA)
            pltpu.sync_copy(mA, g_sp.at[pl.ds(base, B * C)])
            pltpu.sync_copy(g_hbm.at[h, pl.ds(base + B * C, C)],
                            mB.at[pl.ds(0, C)])
            pltpu.sync_copy(mB.at[pl.ds(0, C)],
                            g_sp.at[pl.ds(base + B * C, C)])
            pltpu.sync_copy(zrow_hbm, mA)
            pltpu.sync_copy(mA, acc.at[pl.ds(base, B * C)])
            pltpu.sync_copy(mA.at[pl.ds(0, C)], acc.at[pl.ds(base + B * C, C)])
            plsc.subcore_barrier()

            for c in gath(0, mA):
                c.start()
            for c in gath(0, mA):
                c.wait()

            def body(i, carry):
                for (bi, cur, oth) in ((2 * i, mA, mB), (2 * i + 1, mB, mA)):
                    for c in gath(bi + 1, oth):
                        c.start()
                    for c in scab(bi, cur):
                        c.start(add=True)
                    for c in gath(bi + 1, oth):
                        c.wait()
                    for c in scab(bi, cur):
                        c.wait()
                return carry

            lax.fori_loop(0, nb // 2, body, 0)
            plsc.subcore_barrier()
            pltpu.sync_copy(acc.at[pl.ds(base, B * C)], mA)
            pltpu.sync_copy(mA, out_hbm.at[cid, h, pl.ds(base, B * C)])
            pltpu.sync_copy(acc.at[pl.ds(base + B * C, C)], mB.at[pl.ds(0, C)])
            pltpu.sync_copy(mB.at[pl.ds(0, C)],
                            out_hbm.at[cid, h, pl.ds(base + B * C, C)])

    return scat


@functools.lru_cache(maxsize=None)
def _make_deg(n_pad: int, nchunk: int):
    d = 16
    rows_pt = n_pad // NS
    assert rows_pt == B * C + C
    assert nchunk % 8 == 0 and nchunk >= 24
    mesh = plsc.VectorSubcoreMesh(
        core_axis_name="c", subcore_axis_name="s",
        num_cores=NC, num_subcores=NS)

    @functools.partial(
        pl.kernel,
        out_type=jax.ShapeDtypeStruct((NC, n_pad, d), jnp.float32),
        mesh=mesh,
        scratch_types=[
            pltpu.VMEM((nchunk + DC, C), jnp.int32),
            pltpu.VMEM((C, d), jnp.float32),
            pltpu.VMEM((B * C, d), jnp.float32),
            pltpu.VMEM_SHARED((n_pad, d), jnp.float32),
            pltpu.SemaphoreType.DMA,
        ],
        compiler_params=pltpu.CompilerParams(use_tc_tiling_on_sc=False),
    )
    def deg(dsti_hbm, ones_hbm, zrow_hbm, out_hbm, dsti, ones, stage, acc, ss):
        cid = lax.axis_index("c")
        sid = lax.axis_index("s")
        wid = sid * NC + cid
        base = sid * rows_pt

        def sca(j):
            return pltpu.make_async_copy(ones, acc.at[dsti.at[j]], ss)

        pltpu.sync_copy(dsti_hbm.at[wid], dsti)
        pltpu.sync_copy(ones_hbm, ones)
        pltpu.sync_copy(zrow_hbm, stage)
        pltpu.sync_copy(stage, acc.at[pl.ds(base, B * C)])
        pltpu.sync_copy(stage.at[pl.ds(0, C)], acc.at[pl.ds(base + B * C, C)])
        plsc.subcore_barrier()

        for j in range(16):
            sca(j).start(add=True)

        def body(i, carry):
            j8 = 8 * i
            for k in range(8):
                sca(j8 + k).wait()
            for k in range(8):
                sca(j8 + 16 + k).start(add=True)
            return carry

        lax.fori_loop(0, (nchunk - 16) // 8, body, 0)

        def drain(i, carry):
            for k in range(8):
                sca(nchunk - 16 + 8 * i + k).wait()
            return carry

        lax.fori_loop(0, 2, drain, 0)
        plsc.subcore_barrier()
        pltpu.sync_copy(acc.at[pl.ds(base, B * C)], stage)
        pltpu.sync_copy(stage, out_hbm.at[cid, pl.ds(base, B * C)])
        pltpu.sync_copy(acc.at[pl.ds(base + B * C, C)], stage.at[pl.ds(0, C)])
        pltpu.sync_copy(stage.at[pl.ds(0, C)],
                        out_hbm.at[cid, pl.ds(base + B * C, C)])

    return deg


def _split(gfull, n_pad, d):
    dg = min(d, DG)
    ng = d // dg
    if ng == 1:
        return gfull.reshape(1, n_pad, dg)
    return jnp.stack([gfull[:, h * dg:(h + 1) * dg] for h in range(ng)])


def _joined(acc_ref, g_ref, d):
    dg = min(d, DG)
    ng = d // dg
    parts = [acc_ref[0, h] + acc_ref[1, h] + g_ref[h] for h in range(ng)]
    if ng == 1:
        return parts[0]
    return jnp.concatenate(parts, axis=1)


@functools.lru_cache(maxsize=None)
def _make_tc_mm(n_pad: int, in_dim: int, hid: int):
    def body(x_ref, w_ref, m_ref):
        m_ref[...] = jnp.dot(x_ref[...], w_ref[...],
                             preferred_element_type=jnp.float32)

    return pl.pallas_call(
        body,
        out_shape=jax.ShapeDtypeStruct((n_pad, hid), jnp.float32),
    )


_TC_GRID = 8


@functools.lru_cache(maxsize=None)
def _make_tc_first(n_pad: int, hid: int):
    dg = min(hid, DG)
    ng = hid // dg
    r = n_pad // _TC_GRID

    def body(degp_ref, m_ref, g_ref, dinv_ref):
        deg = degp_ref[0, :, 0:1] + degp_ref[1, :, 0:1] + 1.0
        dinv = lax.rsqrt(deg)
        dinv_ref[...] = jnp.broadcast_to(dinv, (r, 8))
        g_ref[...] = _split(m_ref[...] * dinv, r, hid)

    return pl.pallas_call(
        body,
        grid=(_TC_GRID,),
        in_specs=[
            pl.BlockSpec((NC, r, 16), lambda i: (0, i, 0)),
            pl.BlockSpec((r, hid), lambda i: (i, 0)),
        ],
        out_specs=(
            pl.BlockSpec((ng, r, dg), lambda i: (0, i, 0)),
            pl.BlockSpec((r, 8), lambda i: (i, 0)),
        ),
        out_shape=(
            jax.ShapeDtypeStruct((ng, n_pad, dg), jnp.float32),
            jax.ShapeDtypeStruct((n_pad, 8), jnp.float32),
        ),
    )


@functools.lru_cache(maxsize=None)
def _make_tc_next(n_pad: int, d_in: int, d_out: int):
    dgi = min(d_in, DG)
    ngi = d_in // dgi
    dgo = min(d_out, DG)
    ngo = d_out // dgo
    r = n_pad // _TC_GRID

    def body(acc_ref, g_ref, dinv_ref, b_ref, w_ref, o_ref):
        dv = dinv_ref[:, 0:1]
        conv = dv * _joined(acc_ref, g_ref, d_in) + b_ref[...]
        h = jnp.maximum(conv, 0.0)
        m = jnp.dot(h, w_ref[...], preferred_element_type=jnp.float32)
        o_ref[...] = _split(dv * m, r, d_out)

    return pl.pallas_call(
        body,
        grid=(_TC_GRID,),
        in_specs=[
            pl.BlockSpec((NC, ngi, r, dgi), lambda i: (0, 0, i, 0)),
            pl.BlockSpec((ngi, r, dgi), lambda i: (0, i, 0)),
            pl.BlockSpec((r, 8), lambda i: (i, 0)),
            pl.BlockSpec((1, d_in), lambda i: (0, 0)),
            pl.BlockSpec((d_in, d_out), lambda i: (0, 0)),
        ],
        out_specs=pl.BlockSpec((ngo, r, dgo), lambda i: (0, i, 0)),
        out_shape=jax.ShapeDtypeStruct((ngo, n_pad, dgo), jnp.float32),
    )


@functools.lru_cache(maxsize=None)
def _make_tc_final(n_pad: int, d: int, n_out: int, emb: int):
    def body(acc_ref, g_ref, dinv_ref, b_ref, o_ref):
        dv = dinv_ref[:, 0:1]
        full = dv * _joined(acc_ref, g_ref, d) + b_ref[...]
        o_ref[...] = full[:n_out, :emb]

    return pl.pallas_call(
        body,
        out_shape=jax.ShapeDtypeStruct((n_out, emb), jnp.float32),
    )


def kernel(x, edge_index, W1, b1, W2, b2, W3, b3):
    n, in_dim = x.shape
    e = edge_index.shape[1]
    hid = W1.shape[1]
    emb = W3.shape[1]

    n_pad = -(-n // (NS * (B + 1) * C)) * (NS * (B + 1) * C)
    epw = -(-e // NW)
    nchunk = -(-(-(-epw // C)) // (2 * B)) * (2 * B)
    e_pad = NW * nchunk * C
    trash = jnp.int32(n)

    src = edge_index[0]
    dst = edge_index[1]
    pad = jnp.full((e_pad - e,), trash, dtype=jnp.int32)
    dummy = jnp.full((NW, DC, C), trash, dtype=jnp.int32)
    srcp = jnp.concatenate(
        [jnp.concatenate([src, pad]).reshape(NW, nchunk, C), dummy], axis=1)
    dstp = jnp.concatenate(
        [jnp.concatenate([dst, pad]).reshape(NW, nchunk, C), dummy], axis=1)

    x_pad = jnp.pad(x, ((0, n_pad - n), (0, 0)))
    ones16 = jnp.ones((C, 16), dtype=jnp.float32)
    z16 = jnp.zeros((B * C, 16), dtype=jnp.float32)
    z32 = jnp.zeros((B * C, DG), dtype=jnp.float32)
    emb_p = 16
    W3p = jnp.pad(W3, ((0, 0), (0, emb_p - emb)))
    b3p = jnp.pad(b3, (0, emb_p - emb)).reshape(1, emb_p)
    b1r = b1.reshape(1, hid)
    b2r = b2.reshape(1, hid)

    scat16 = _make_scatter(n_pad, nchunk, emb_p)
    scath = _make_scatter(n_pad, nchunk, hid)

    degp = _make_deg(n_pad, nchunk)(dstp, ones16, z16)
    m1 = _make_tc_mm(n_pad, in_dim, hid)(x_pad, W1)
    g1, dinv = _make_tc_first(n_pad, hid)(degp, m1)
    acc1 = scath(g1, srcp, dstp, z32)
    g2 = _make_tc_next(n_pad, hid, hid)(acc1, g1, dinv, b1r, W2)
    acc2 = scath(g2, srcp, dstp, z32)
    g3 = _make_tc_next(n_pad, hid, emb_p)(acc2, g2, dinv, b2r, W3p)
    acc3 = scat16(g3, srcp, dstp, z16)
    return _make_tc_final(n_pad, emb_p, n, emb)(acc3, g3, dinv, b3p)


# speedup vs baseline: 1.0553x; 1.0116x over previous
import functools

import jax
import jax.numpy as jnp
from jax import lax
from jax.experimental import pallas as pl
from jax.experimental.pallas import tpu as pltpu
from jax.experimental.pallas import tpu_sc as plsc

NC = 2
NS = 16
NW = NC * NS
C = 128
B = 4
DC = 4
DG = 32


@functools.lru_cache(maxsize=None)
def _make_scatter(n_pad: int, nchunk: int, d: int):
    dg = min(d, DG)
    ng = d // dg
    assert dg * ng == d
    assert nchunk % (2 * B) == 0
    nb = nchunk // B
    rows_pt = n_pad // NS
    assert rows_pt == B * C + C
    mesh = plsc.VectorSubcoreMesh(
        core_axis_name="c", subcore_axis_name="s",
        num_cores=NC, num_subcores=NS)

    @functools.partial(
        pl.kernel,
        out_type=jax.ShapeDtypeStruct((NC, ng, n_pad, dg), jnp.float32),
        mesh=mesh,
        scratch_types=[
            pltpu.VMEM((nchunk + DC, C), jnp.int32),
            pltpu.VMEM((nchunk + DC, C), jnp.int32),
            pltpu.VMEM((B * C, dg), jnp.float32),
            pltpu.VMEM((B * C, dg), jnp.float32),
            pltpu.VMEM_SHARED((n_pad, dg), jnp.float32),
            pltpu.VMEM_SHARED((n_pad, dg), jnp.float32),
            pltpu.SemaphoreType.DMA,
        ],
        compiler_params=pltpu.CompilerParams(use_tc_tiling_on_sc=False),
    )
    def scat(g_hbm, srci_hbm, dsti_hbm, zrow_hbm, out_hbm,
             srci, dsti, mA, mB, acc, g_sp, s):
        cid = lax.axis_index("c")
        sid = lax.axis_index("s")
        wid = sid * NC + cid
        base = sid * rows_pt

        def gath(bi, buf):
            return [pltpu.make_async_copy(
                        g_sp.at[srci.at[B * bi + k]],
                        buf.at[pl.ds(k * C, C)], s)
                    for k in range(B)]

        def scab(bi, buf):
            return [pltpu.make_async_copy(
                        buf.at[pl.ds(k * C, C)],
                        acc.at[dsti.at[B * bi + k]], s)
                    for k in range(B)]

        pltpu.sync_copy(srci_hbm.at[wid], srci)
        pltpu.sync_copy(dsti_hbm.at[wid], dsti)

        for h in range(ng):
            pltpu.sync_copy(g_hbm.at[h, pl.ds(base, B * C)], mA)
            pltpu.sync_copy(mA, g_sp.at[pl.ds(base, B * C)])
            pltpu.sync_copy(g_hbm.at[h, pl.ds(base + B * C, C)],
                            mB.at[pl.ds(0, C)])
            pltpu.sync_copy(mB.at[pl.ds(0, C)],
                            g_sp.at[pl.ds(base + B * C, C)])
            pltpu.sync_copy(zrow_hbm, mA)
            pltpu.sync_copy(mA, acc.at[pl.ds(base, B * C)])
            pltpu.sync_copy(mA.at[pl.ds(0, C)], acc.at[pl.ds(base + B * C, C)])
            plsc.subcore_barrier()

            for c in gath(0, mA):
                c.start()
            for c in gath(0, mA):
                c.wait()
            for c in scab(0, mA):
                c.start(add=True)
            for c in gath(1, mB):
                c.start()

            def body(i, carry):
                for (r, cur, oth) in ((2 * i + 1, mB, mA),
                                      (2 * i + 2, mA, mB)):
                    for c in gath(r, cur):
                        c.wait()
                    for c in scab(r - 1, oth):
                        c.wait()
                    for c in scab(r, cur):
                        c.start(add=True)
                    for c in gath(r + 1, oth):
                        c.start()
                return carry

            lax.fori_loop(0, (nb - 2) // 2, body, 0)
            for c in gath(nb - 1, mB):
                c.wait()
            for c in scab(nb - 2, mA):
                c.wait()
            for c in scab(nb - 1, mB):
                c.start(add=True)
            for c in scab(nb - 1, mB):
                c.wait()
            plsc.subcore_barrier()
            pltpu.sync_copy(acc.at[pl.ds(base, B * C)], mA)
            pltpu.sync_copy(mA, out_hbm.at[cid, h, pl.ds(base, B * C)])
            pltpu.sync_copy(acc.at[pl.ds(base + B * C, C)], mB.at[pl.ds(0, C)])
            pltpu.sync_copy(mB.at[pl.ds(0, C)],
                            out_hbm.at[cid, h, pl.ds(base + B * C, C)])

    return scat


@functools.lru_cache(maxsize=None)
def _make_deg(n_pad: int, nchunk: int):
    d = 16
    rows_pt = n_pad // NS
    assert rows_pt == B * C + C
    assert nchunk % 8 == 0 and nchunk >= 24
    mesh = plsc.VectorSubcoreMesh(
        core_axis_name="c", subcore_axis_name="s",
        num_cores=NC, num_subcores=NS)

    @functools.partial(
        pl.kernel,
        out_type=jax.ShapeDtypeStruct((NC, n_pad, d), jnp.float32),
        mesh=mesh,
        scratch_types=[
            pltpu.VMEM((nchunk + DC, C), jnp.int32),
            pltpu.VMEM((C, d), jnp.float32),
            pltpu.VMEM((B * C, d), jnp.float32),
            pltpu.VMEM_SHARED((n_pad, d), jnp.float32),
            pltpu.SemaphoreType.DMA,
        ],
        compiler_params=pltpu.CompilerParams(use_tc_tiling_on_sc=False),
    )
    def deg(dsti_hbm, ones_hbm, zrow_hbm, out_hbm, dsti, ones, stage, acc, ss):
        cid = lax.axis_index("c")
        sid = lax.axis_index("s")
        wid = sid * NC + cid
        base = sid * rows_pt

        def sca(j):
            return pltpu.make_async_copy(ones, acc.at[dsti.at[j]], ss)

        pltpu.sync_copy(dsti_hbm.at[wid], dsti)
        pltpu.sync_copy(ones_hbm, ones)
        pltpu.sync_copy(zrow_hbm, stage)
        pltpu.sync_copy(stage, acc.at[pl.ds(base, B * C)])
        pltpu.sync_copy(stage.at[pl.ds(0, C)], acc.at[pl.ds(base + B * C, C)])
        plsc.subcore_barrier()

        for j in range(16):
            sca(j).start(add=True)

        def body(i, carry):
            j8 = 8 * i
            for k in range(8):
                sca(j8 + k).wait()
            for k in range(8):
                sca(j8 + 16 + k).start(add=True)
            return carry

        lax.fori_loop(0, (nchunk - 16) // 8, body, 0)

        def drain(i, carry):
            for k in range(8):
                sca(nchunk - 16 + 8 * i + k).wait()
            return carry

        lax.fori_loop(0, 2, drain, 0)
        plsc.subcore_barrier()
        pltpu.sync_copy(acc.at[pl.ds(base, B * C)], stage)
        pltpu.sync_copy(stage, out_hbm.at[cid, pl.ds(base, B * C)])
        pltpu.sync_copy(acc.at[pl.ds(base + B * C, C)], stage.at[pl.ds(0, C)])
        pltpu.sync_copy(stage.at[pl.ds(0, C)],
                        out_hbm.at[cid, pl.ds(base + B * C, C)])

    return deg


def _split(gfull, n_pad, d):
    dg = min(d, DG)
    ng = d // dg
    if ng == 1:
        return gfull.reshape(1, n_pad, dg)
    return jnp.stack([gfull[:, h * dg:(h + 1) * dg] for h in range(ng)])


def _joined(acc_ref, g_ref, d):
    dg = min(d, DG)
    ng = d // dg
    parts = [acc_ref[0, h] + acc_ref[1, h] + g_ref[h] for h in range(ng)]
    if ng == 1:
        return parts[0]
    return jnp.concatenate(parts, axis=1)


@functools.lru_cache(maxsize=None)
def _make_tc_mm(n_pad: int, in_dim: int, hid: int):
    def body(x_ref, w_ref, m_ref):
        m_ref[...] = jnp.dot(x_ref[...], w_ref[...],
                             preferred_element_type=jnp.float32)

    return pl.pallas_call(
        body,
        out_shape=jax.ShapeDtypeStruct((n_pad, hid), jnp.float32),
    )


_TC_GRID = 8


@functools.lru_cache(maxsize=None)
def _make_tc_first(n_pad: int, hid: int):
    dg = min(hid, DG)
    ng = hid // dg
    r = n_pad // _TC_GRID

    def body(degp_ref, m_ref, g_ref, dinv_ref):
        deg = degp_ref[0, :, 0:1] + degp_ref[1, :, 0:1] + 1.0
        dinv = lax.rsqrt(deg)
        dinv_ref[...] = jnp.broadcast_to(dinv, (r, 8))
        g_ref[...] = _split(m_ref[...] * dinv, r, hid)

    return pl.pallas_call(
        body,
        grid=(_TC_GRID,),
        in_specs=[
            pl.BlockSpec((NC, r, 16), lambda i: (0, i, 0)),
            pl.BlockSpec((r, hid), lambda i: (i, 0)),
        ],
        out_specs=(
            pl.BlockSpec((ng, r, dg), lambda i: (0, i, 0)),
            pl.BlockSpec((r, 8), lambda i: (i, 0)),
        ),
        out_shape=(
            jax.ShapeDtypeStruct((ng, n_pad, dg), jnp.float32),
            jax.ShapeDtypeStruct((n_pad, 8), jnp.float32),
        ),
    )


@functools.lru_cache(maxsize=None)
def _make_tc_next(n_pad: int, d_in: int, d_out: int):
    dgi = min(d_in, DG)
    ngi = d_in // dgi
    dgo = min(d_out, DG)
    ngo = d_out // dgo
    r = n_pad // _TC_GRID

    def body(acc_ref, g_ref, dinv_ref, b_ref, w_ref, o_ref):
        dv = dinv_ref[:, 0:1]
        conv = dv * _joined(acc_ref, g_ref, d_in) + b_ref[...]
        h = jnp.maximum(conv, 0.0)
        m = jnp.dot(h, w_ref[...], preferred_element_type=jnp.float32)
        o_ref[...] = _split(dv * m, r, d_out)

    return pl.pallas_call(
        body,
        grid=(_TC_GRID,),
        in_specs=[
            pl.BlockSpec((NC, ngi, r, dgi), lambda i: (0, 0, i, 0)),
            pl.BlockSpec((ngi, r, dgi), lambda i: (0, i, 0)),
            pl.BlockSpec((r, 8), lambda i: (i, 0)),
            pl.BlockSpec((1, d_in), lambda i: (0, 0)),
            pl.BlockSpec((d_in, d_out), lambda i: (0, 0)),
        ],
        out_specs=pl.BlockSpec((ngo, r, dgo), lambda i: (0, i, 0)),
        out_shape=jax.ShapeDtypeStruct((ngo, n_pad, dgo), jnp.float32),
    )


@functools.lru_cache(maxsize=None)
def _make_tc_final(n_pad: int, d: int, n_out: int, emb: int):
    def body(acc_ref, g_ref, dinv_ref, b_ref, o_ref):
        dv = dinv_ref[:, 0:1]
        full = dv * _joined(acc_ref, g_ref, d) + b_ref[...]
        o_ref[...] = full[:n_out, :emb]

    return pl.pallas_call(
        body,
        out_shape=jax.ShapeDtypeStruct((n_out, emb), jnp.float32),
    )


def kernel(x, edge_index, W1, b1, W2, b2, W3, b3):
    n, in_dim = x.shape
    e = edge_index.shape[1]
    hid = W1.shape[1]
    emb = W3.shape[1]

    n_pad = -(-n // (NS * (B + 1) * C)) * (NS * (B + 1) * C)
    epw = -(-e // NW)
    nchunk = -(-(-(-epw // C)) // (2 * B)) * (2 * B)
    e_pad = NW * nchunk * C
    trash = jnp.int32(n)

    src = edge_index[0]
    dst = edge_index[1]
    pad = jnp.full((e_pad - e,), trash, dtype=jnp.int32)
    dummy = jnp.full((NW, DC, C), trash, dtype=jnp.int32)
    srcp = jnp.concatenate(
        [jnp.concatenate([src, pad]).reshape(NW, nchunk, C), dummy], axis=1)
    dstp = jnp.concatenate(
        [jnp.concatenate([dst, pad]).reshape(NW, nchunk, C), dummy], axis=1)

    x_pad = jnp.pad(x, ((0, n_pad - n), (0, 0)))
    ones16 = jnp.ones((C, 16), dtype=jnp.float32)
    z16 = jnp.zeros((B * C, 16), dtype=jnp.float32)
    z32 = jnp.zeros((B * C, DG), dtype=jnp.float32)
    emb_p = 16
    W3p = jnp.pad(W3, ((0, 0), (0, emb_p - emb)))
    b3p = jnp.pad(b3, (0, emb_p - emb)).reshape(1, emb_p)
    b1r = b1.reshape(1, hid)
    b2r = b2.reshape(1, hid)

    scat16 = _make_scatter(n_pad, nchunk, emb_p)
    scath = _make_scatter(n_pad, nchunk, hid)

    degp = _make_deg(n_pad, nchunk)(dstp, ones16, z16)
    m1 = _make_tc_mm(n_pad, in_dim, hid)(x_pad, W1)
    g1, dinv = _make_tc_first(n_pad, hid)(degp, m1)
    acc1 = scath(g1, srcp, dstp, z32)
    g2 = _make_tc_next(n_pad, hid, hid)(acc1, g1, dinv, b1r, W2)
    acc2 = scath(g2, srcp, dstp, z32)
    g3 = _make_tc_next(n_pad, hid, emb_p)(acc2, g2, dinv, b2r, W3p)
    acc3 = scat16(g3, srcp, dstp, z16)
    return _make_tc_final(n_pad, emb_p, n, emb)(acc3, g3, dinv, b3p)
